# baseline (speedup 1.0000x reference)
import functools
import math

import jax
import jax.numpy as jnp
import numpy as np
from jax import lax
from jax.experimental import pallas as pl
from jax.experimental.pallas import tpu as pltpu
from jax.experimental.pallas import tpu_sc as plsc

SMOOTHING = 0.1
CONFIDENCE = 1.0 - SMOOTHING


def _rowsum_kernel(x_ref, m_ref, acc_ref, msum_ref):
    i = pl.program_id(0)
    j = pl.program_id(1)

    @pl.when((i == 0) & (j == 0))
    def _init():
        acc_ref[...] = jnp.zeros((1, 1), jnp.float32)
        msum_ref[...] = jnp.zeros((1, 1), jnp.float32)

    xb = x_ref[...]
    rows = xb.shape[0]
    mb = m_ref[0, 0, :]
    rsum = jnp.sum(xb, axis=1)
    acc_ref[...] += jnp.sum(rsum * mb).reshape(1, 1)

    @pl.when(j == 0)
    def _msum():
        msum_ref[...] += jnp.sum(mb).reshape(1, 1)


def _make_sc_gather(N, V, n_workers, rows_per_w):
    mesh = plsc.VectorSubcoreMesh(core_axis_name="c", subcore_axis_name="s")
    NC = 2
    L = 16
    n_chunks = rows_per_w // L

    @functools.partial(
        pl.kernel,
        mesh=mesh,
        out_type=jax.ShapeDtypeStruct((n_workers, L), jnp.float32),
        scratch_types=[
            pltpu.VMEM((rows_per_w,), jnp.int32),
            pltpu.VMEM((rows_per_w,), jnp.float32),
            pltpu.VMEM((rows_per_w,), jnp.int32),
            pltpu.VMEM((rows_per_w,), jnp.float32),
            pltpu.VMEM((L,), jnp.float32),
            pltpu.SemaphoreType.DMA,
        ],
    )
    def sc_gather(x_hbm, t_hbm, m_hbm, out_hbm, t_v, m_v, idx_v, g_v,
                  acc_v, sem):
        wid = lax.axis_index("s") * NC + lax.axis_index("c")
        base = wid * rows_per_w
        pltpu.sync_copy(t_hbm.at[pl.ds(base, rows_per_w)], t_v)
        pltpu.sync_copy(m_hbm.at[pl.ds(base, rows_per_w)], m_v)
        iota = lax.iota(jnp.int32, L)
        for k in range(n_chunks):
            row_ids = base + k * L + iota
            idx_v[pl.ds(k * L, L)] = row_ids * V + t_v[pl.ds(k * L, L)]
        pltpu.async_copy(x_hbm.at[idx_v], g_v, sem).wait()
        acc = jnp.zeros((L,), jnp.float32)
        for k in range(n_chunks):
            acc = acc + g_v[pl.ds(k * L, L)] * m_v[pl.ds(k * L, L)]
        acc_v[...] = acc
        pltpu.sync_copy(acc_v, out_hbm.at[wid])

    return sc_gather


def kernel(input, target, mask):
    B, T, V = input.shape
    N = B * T
    x = input.reshape(N, V)
    t = target.reshape(N).astype(jnp.int32)
    m = mask.reshape(N).astype(jnp.float32)

    fill = float(np.float32(SMOOTHING / (V - 1)))
    conf = CONFIDENCE
    c_const = (V - 1) * fill * math.log(fill) + conf * math.log(conf)

    NW = 32
    sc_gather = _make_sc_gather(N, V, NW, N // NW)
    gpart = sc_gather(input.reshape(N * V), t, m)

    ROWS = 1024
    COLS = 4096
    n_i = N // ROWS
    n_j = V // COLS
    m3 = m.reshape(n_i, 1, ROWS)

    acc, msum = pl.pallas_call(
        _rowsum_kernel,
        grid=(n_i, n_j),
        in_specs=[
            pl.BlockSpec((ROWS, COLS), lambda i, j: (i, j)),
            pl.BlockSpec((1, 1, ROWS), lambda i, j: (i, 0, 0)),
        ],
        out_specs=[
            pl.BlockSpec((1, 1), lambda i, j: (0, 0)),
            pl.BlockSpec((1, 1), lambda i, j: (0, 0)),
        ],
        out_shape=[
            jax.ShapeDtypeStruct((1, 1), jnp.float32),
            jax.ShapeDtypeStruct((1, 1), jnp.float32),
        ],
    )(x, m3)

    g = jnp.sum(gpart)
    return (jnp.float32(c_const)
            - (fill * acc[0, 0] + (conf - fill) * g) / msum[0, 0])


# speedup vs baseline: 2.5285x; 2.5285x over previous
import functools
import math

import jax
import jax.numpy as jnp
import numpy as np
from jax import lax
from jax.experimental import pallas as pl
from jax.experimental.pallas import tpu as pltpu
from jax.experimental.pallas import tpu_sc as plsc

SMOOTHING = 0.1
CONFIDENCE = 1.0 - SMOOTHING

N_SC = 512
SC_CW = 4096
TC_ROWS = 512
TC_COLS = 4096


def _tc_kernel(x_ref, t_ref, m_ref, acc_ref, msum_ref, *, cols_per_blk,
               fill, conf):
    i = pl.program_id(0)
    j = pl.program_id(1)

    @pl.when((i == 0) & (j == 0))
    def _init():
        acc_ref[...] = jnp.zeros((1, 1), jnp.float32)
        msum_ref[...] = jnp.zeros((1, 1), jnp.float32)

    xb = x_ref[...]
    rows = xb.shape[0]
    tb = t_ref[0, 0, :].reshape(rows, 1)
    mb = m_ref[0, 0, :]

    tloc = tb - j * cols_per_blk
    cols = jax.lax.broadcasted_iota(jnp.int32, xb.shape, 1)
    sel = cols == tloc
    rsum = jnp.sum(xb, axis=1)
    gsum = jnp.sum(jnp.where(sel, xb, 0.0), axis=1)
    rowpart = fill * rsum + (conf - fill) * gsum
    acc_ref[...] += jnp.sum(rowpart * mb).reshape(1, 1)

    @pl.when(j == 0)
    def _msum():
        msum_ref[...] += jnp.sum(mb).reshape(1, 1)


def _make_sc_kernel(V, fill, dconf):
    mesh = plsc.VectorSubcoreMesh(core_axis_name="c", subcore_axis_name="s")
    NC = 2
    L = 16
    NW = 32
    rows_per_w = N_SC // NW
    n_groups = rows_per_w // 8
    n_chunks = V // SC_CW
    shift = int(math.log2(SC_CW))
    n_segs = SC_CW // L

    @functools.partial(
        pl.kernel,
        mesh=mesh,
        out_type=jax.ShapeDtypeStruct((NW, rows_per_w + 1, L), jnp.float32),
        scratch_types=[
            pltpu.VMEM((8 * SC_CW,), jnp.float32),
            pltpu.VMEM((8 * SC_CW,), jnp.float32),
            pltpu.VMEM((L,), jnp.int32),
            pltpu.VMEM((L,), jnp.float32),
            pltpu.VMEM((rows_per_w + 1, L), jnp.float32),
            pltpu.SemaphoreType.DMA,
            pltpu.SemaphoreType.DMA,
        ],
    )
    def sc_kernel(x_hbm, t_hbm, m_hbm, out_hbm, buf_a, buf_b, t_v, m_v,
                  stage_v, sem_a, sem_b):
        wid = lax.axis_index("s") * NC + lax.axis_index("c")
        row0 = wid * rows_per_w
        pltpu.sync_copy(t_hbm.at[pl.ds(row0, rows_per_w)], t_v)
        pltpu.sync_copy(m_hbm.at[pl.ds(row0, rows_per_w)], m_v)

        iota = lax.iota(jnp.int32, L)
        tv = t_v[...]
        mv = m_v[...]
        zeros = jnp.zeros((L,), jnp.float32)

        bufs = (buf_a, buf_b)
        sems = (sem_a, sem_b)

        for g in range(n_groups):
            gbase = row0 + g * 8
            lane_lo = g * 8
            t_i = [tv[lane_lo + i] for i in range(8)]
            t_chunk = [lax.shift_right_logical(t, shift) for t in t_i]
            t_mod = [t & (SC_CW - 1) for t in t_i]

            racc = tuple(zeros for _ in range(8))
            gvec = [zeros for _ in range(8)]
            pending = ()
            for k in range(n_chunks + 1):
                cur = ()
                if k < n_chunks:
                    cur = tuple(
                        pltpu.async_copy(
                            x_hbm.at[gbase + i, pl.ds(k * SC_CW, SC_CW)],
                            bufs[k % 2].at[pl.ds(i * SC_CW, SC_CW)],
                            sems[k % 2])
                        for i in range(8))
                if pending:
                    kp = k - 1
                    for c in pending:
                        c.wait()
                    buf = bufs[kp % 2]

                    def seg_body(s, carry):
                        base = s * L
                        return tuple(
                            carry[i] + buf[pl.ds(i * SC_CW + base, L)]
                            for i in range(8))

                    racc = lax.fori_loop(0, n_segs, seg_body, racc)

                    for i in range(8):
                        seg = buf[pl.ds(i * SC_CW + (t_mod[i] & ~(L - 1)),
                                        L)]
                        lane_eq = iota == (t_mod[i] & (L - 1))
                        flag = jnp.where(t_chunk[i] == kp,
                                         jnp.float32(1.0), jnp.float32(0.0))
                        gvec[i] = gvec[i] + jnp.where(lane_eq, seg, 0.0) * flag
                pending = cur

            for i in range(8):
                m_i = mv[lane_lo + i]
                stage_v[lane_lo + i, :] = (
                    (fill * racc[i] + dconf * gvec[i]) * m_i)

        stage_v[rows_per_w, :] = mv
        pltpu.sync_copy(stage_v, out_hbm.at[wid])

    return sc_kernel


def kernel(input, target, mask):
    B, T, V = input.shape
    N = B * T
    x = input.reshape(N, V)
    t = target.reshape(N).astype(jnp.int32)
    m = mask.reshape(N).astype(jnp.float32)

    fill = float(np.float32(SMOOTHING / (V - 1)))
    conf = CONFIDENCE
    dconf = float(np.float32(conf - fill))
    c_const = (V - 1) * fill * math.log(fill) + conf * math.log(conf)

    sc_kernel = _make_sc_kernel(V, fill, dconf)
    sc_out = sc_kernel(x, t, m)

    n_i = (N - N_SC) // TC_ROWS
    n_j = V // TC_COLS
    i_off = N_SC // TC_ROWS
    t3 = t.reshape(N // TC_ROWS, 1, TC_ROWS)
    m3 = m.reshape(N // TC_ROWS, 1, TC_ROWS)

    body = functools.partial(_tc_kernel, cols_per_blk=TC_COLS,
                             fill=fill, conf=conf)
    acc, msum = pl.pallas_call(
        body,
        grid=(n_i, n_j),
        in_specs=[
            pl.BlockSpec((TC_ROWS, TC_COLS), lambda i, j: (i + i_off, j)),
            pl.BlockSpec((1, 1, TC_ROWS), lambda i, j: (i + i_off, 0, 0)),
            pl.BlockSpec((1, 1, TC_ROWS), lambda i, j: (i + i_off, 0, 0)),
        ],
        out_specs=[
            pl.BlockSpec((1, 1), lambda i, j: (0, 0)),
            pl.BlockSpec((1, 1), lambda i, j: (0, 0)),
        ],
        out_shape=[
            jax.ShapeDtypeStruct((1, 1), jnp.float32),
            jax.ShapeDtypeStruct((1, 1), jnp.float32),
        ],
    )(x, t3, m3)

    g_sc = jnp.sum(sc_out[:, :-1, :])
    msum_all = msum[0, 0] + jnp.sum(sc_out[:, -1, :])
    return jnp.float32(c_const) - (acc[0, 0] + g_sc) / msum_all


# speedup vs baseline: 2.5361x; 1.0030x over previous
import functools
import math

import jax
import jax.numpy as jnp
import numpy as np
from jax import lax
from jax.experimental import pallas as pl
from jax.experimental.pallas import tpu as pltpu
from jax.experimental.pallas import tpu_sc as plsc

SMOOTHING = 0.1
CONFIDENCE = 1.0 - SMOOTHING

N_SC = 512
SC_CW = 4096
TC_ROWS = 512
TC_COLS = 4096


def _tc_kernel(x_ref, t_ref, m_ref, acc_ref, msum_ref, *, cols_per_blk,
               fill, conf):
    i = pl.program_id(0)
    j = pl.program_id(1)

    @pl.when((i == 0) & (j == 0))
    def _init():
        acc_ref[...] = jnp.zeros((1, 1), jnp.float32)
        msum_ref[...] = jnp.zeros((1, 1), jnp.float32)

    xb = x_ref[...]
    rows = xb.shape[0]
    tb = t_ref[0, 0, :].reshape(rows, 1)
    mb = m_ref[0, 0, :]

    tloc = tb - j * cols_per_blk
    cols = jax.lax.broadcasted_iota(jnp.int32, xb.shape, 1)
    sel = cols == tloc
    rsum = jnp.sum(xb, axis=1)
    gsum = jnp.sum(jnp.where(sel, xb, 0.0), axis=1)
    rowpart = fill * rsum + (conf - fill) * gsum
    acc_ref[...] += jnp.sum(rowpart * mb).reshape(1, 1)

    @pl.when(j == 0)
    def _msum():
        msum_ref[...] += jnp.sum(mb).reshape(1, 1)


def _make_sc_kernel(V, fill, dconf):
    mesh = plsc.VectorSubcoreMesh(core_axis_name="c", subcore_axis_name="s")
    NC = 2
    L = 16
    NW = 32
    rows_per_w = N_SC // NW
    n_groups = rows_per_w // 8
    n_chunks = V // SC_CW
    shift = int(math.log2(SC_CW))
    n_segs = SC_CW // L

    @functools.partial(
        pl.kernel,
        mesh=mesh,
        out_type=jax.ShapeDtypeStruct((NW, rows_per_w + 1, L), jnp.float32),
        scratch_types=[
            pltpu.VMEM((8 * SC_CW,), jnp.float32),
            pltpu.VMEM((8 * SC_CW,), jnp.float32),
            pltpu.VMEM((L,), jnp.int32),
            pltpu.VMEM((L,), jnp.float32),
            pltpu.VMEM((rows_per_w + 1, L), jnp.float32),
            pltpu.SemaphoreType.DMA,
            pltpu.SemaphoreType.DMA,
        ],
    )
    def sc_kernel(x_hbm, t_hbm, m_hbm, out_hbm, buf_a, buf_b, t_v, m_v,
                  stage_v, sem_a, sem_b):
        wid = lax.axis_index("s") * NC + lax.axis_index("c")
        row0 = wid * rows_per_w
        pltpu.sync_copy(t_hbm.at[pl.ds(row0, rows_per_w)], t_v)
        pltpu.sync_copy(m_hbm.at[pl.ds(row0, rows_per_w)], m_v)

        iota = lax.iota(jnp.int32, L)
        tv = t_v[...]
        mv = m_v[...]
        zeros = jnp.zeros((L,), jnp.float32)

        bufs = (buf_a, buf_b)
        sems = (sem_a, sem_b)

        for g in range(n_groups):
            gbase = row0 + g * 8
            lane_lo = g * 8
            t_i = [tv[lane_lo + i] for i in range(8)]
            t_chunk = [lax.shift_right_logical(t, shift) for t in t_i]
            t_mod = [t & (SC_CW - 1) for t in t_i]

            racc = tuple(zeros for _ in range(8))
            gvec = [zeros for _ in range(8)]
            pending = ()
            for k in range(n_chunks + 1):
                cur = ()
                if k < n_chunks:
                    cur = tuple(
                        pltpu.async_copy(
                            x_hbm.at[gbase + i, pl.ds(k * SC_CW, SC_CW)],
                            bufs[k % 2].at[pl.ds(i * SC_CW, SC_CW)],
                            sems[k % 2])
                        for i in range(8))
                if pending:
                    kp = k - 1
                    for c in pending:
                        c.wait()
                    buf = bufs[kp % 2]

                    def seg_body(s, carry):
                        base = s * L
                        return tuple(
                            carry[i] + buf[pl.ds(i * SC_CW + base, L)]
                            for i in range(8))

                    racc = lax.fori_loop(0, n_segs, seg_body, racc)

                    for i in range(8):
                        seg = buf[pl.ds(i * SC_CW + (t_mod[i] & ~(L - 1)),
                                        L)]
                        lane_eq = iota == (t_mod[i] & (L - 1))
                        flag = jnp.where(t_chunk[i] == kp,
                                         jnp.float32(1.0), jnp.float32(0.0))
                        gvec[i] = gvec[i] + jnp.where(lane_eq, seg, 0.0) * flag
                pending = cur

            for i in range(8):
                m_i = mv[lane_lo + i]
                stage_v[lane_lo + i, :] = (
                    (fill * racc[i] + dconf * gvec[i]) * m_i)

        stage_v[rows_per_w, :] = mv
        pltpu.sync_copy(stage_v, out_hbm.at[wid])

    return sc_kernel


def kernel(input, target, mask):
    B, T, V = input.shape
    N = B * T
    x = input.reshape(N, V)
    t = target.reshape(N).astype(jnp.int32)
    m = mask.reshape(N).astype(jnp.float32)

    fill = float(np.float32(SMOOTHING / (V - 1)))
    conf = CONFIDENCE
    dconf = float(np.float32(conf - fill))
    c_const = (V - 1) * fill * math.log(fill) + conf * math.log(conf)

    n_i = (N - N_SC) // TC_ROWS
    n_j = V // TC_COLS
    i_off = N_SC // TC_ROWS
    t3 = t.reshape(N // TC_ROWS, 1, TC_ROWS)
    m3 = m.reshape(N // TC_ROWS, 1, TC_ROWS)

    body = functools.partial(_tc_kernel, cols_per_blk=TC_COLS,
                             fill=fill, conf=conf)
    acc, msum = pl.pallas_call(
        body,
        grid=(n_i, n_j),
        in_specs=[
            pl.BlockSpec((TC_ROWS, TC_COLS), lambda i, j: (i + i_off, j)),
            pl.BlockSpec((1, 1, TC_ROWS), lambda i, j: (i + i_off, 0, 0)),
            pl.BlockSpec((1, 1, TC_ROWS), lambda i, j: (i + i_off, 0, 0)),
        ],
        out_specs=[
            pl.BlockSpec((1, 1), lambda i, j: (0, 0)),
            pl.BlockSpec((1, 1), lambda i, j: (0, 0)),
        ],
        out_shape=[
            jax.ShapeDtypeStruct((1, 1), jnp.float32),
            jax.ShapeDtypeStruct((1, 1), jnp.float32),
        ],
    )(x, t3, m3)

    sc_kernel = _make_sc_kernel(V, fill, dconf)
    sc_out = sc_kernel(x, t, m)

    g_sc = jnp.sum(sc_out[:, :-1, :])
    msum_all = msum[0, 0] + jnp.sum(sc_out[:, -1, :])
    return jnp.float32(c_const) - (acc[0, 0] + g_sc) / msum_all


# speedup vs baseline: 2.7581x; 1.0875x over previous
import functools
import math

import jax
import jax.numpy as jnp
import numpy as np
from jax import lax
from jax.experimental import pallas as pl
from jax.experimental.pallas import tpu as pltpu
from jax.experimental.pallas import tpu_sc as plsc

SMOOTHING = 0.1
CONFIDENCE = 1.0 - SMOOTHING

TC_ROWS = 1024
TC_COLS = 4096


def _tc_kernel(x_ref, m_ref, acc_ref, msum_ref):
    i = pl.program_id(0)
    j = pl.program_id(1)

    @pl.when((i == 0) & (j == 0))
    def _init():
        acc_ref[...] = jnp.zeros((1, 1), jnp.float32)
        msum_ref[...] = jnp.zeros((1, 1), jnp.float32)

    xb = x_ref[...]
    mb = m_ref[0, 0, :]
    rsum = jnp.sum(xb, axis=1)
    acc_ref[...] += jnp.sum(rsum * mb).reshape(1, 1)

    @pl.when(j == 0)
    def _msum():
        msum_ref[...] += jnp.sum(mb).reshape(1, 1)


def _make_sc_gather(N, V):
    mesh = plsc.VectorSubcoreMesh(core_axis_name="c", subcore_axis_name="s")
    NC = 2
    L = 16
    NW = 32
    rows_per_w = N // NW
    n_chunks = rows_per_w // L

    @functools.partial(
        pl.kernel,
        mesh=mesh,
        out_type=jax.ShapeDtypeStruct((NW, L), jnp.float32),
        scratch_types=[
            pltpu.VMEM((rows_per_w,), jnp.int32),
            pltpu.VMEM((rows_per_w,), jnp.float32),
            pltpu.VMEM((rows_per_w * L,), jnp.float32),
            pltpu.VMEM((L,), jnp.float32),
            pltpu.SemaphoreType.DMA,
        ],
    )
    def sc_gather(x_hbm, t_hbm, m_hbm, out_hbm, t_v, m_v, seg_v, stage_v,
                  sem):
        wid = lax.axis_index("s") * NC + lax.axis_index("c")
        row0 = wid * rows_per_w
        pltpu.sync_copy(t_hbm.at[pl.ds(row0, rows_per_w)], t_v)
        pltpu.sync_copy(m_hbm.at[pl.ds(row0, rows_per_w)], m_v)

        iota = lax.iota(jnp.int32, L)
        acc = jnp.zeros((L,), jnp.float32)

        for c in range(n_chunks):
            tv = t_v[pl.ds(c * L, L)]
            mv = m_v[pl.ds(c * L, L)]
            t_i = [tv[i] for i in range(L)]
            copies = []
            for i in range(L):
                r = c * L + i
                c0 = pl.multiple_of(t_i[i] & ~(L - 1), L)
                copies.append(pltpu.async_copy(
                    x_hbm.at[row0 + r, pl.ds(c0, L)],
                    seg_v.at[pl.ds(r * L, L)], sem))
            for cp in copies:
                cp.wait()
            for i in range(L):
                r = c * L + i
                seg = seg_v[pl.ds(r * L, L)]
                hit = iota == (t_i[i] & (L - 1))
                acc = acc + jnp.where(hit, seg, 0.0) * mv[i]

        stage_v[...] = acc
        pltpu.sync_copy(stage_v, out_hbm.at[wid])

    return sc_gather


def kernel(input, target, mask):
    B, T, V = input.shape
    N = B * T
    x = input.reshape(N, V)
    t = target.reshape(N).astype(jnp.int32)
    m = mask.reshape(N).astype(jnp.float32)

    fill = float(np.float32(SMOOTHING / (V - 1)))
    conf = CONFIDENCE
    dconf = float(np.float32(conf - fill))
    c_const = (V - 1) * fill * math.log(fill) + conf * math.log(conf)

    n_i = N // TC_ROWS
    n_j = V // TC_COLS
    m3 = m.reshape(n_i, 1, TC_ROWS)
    acc, msum = pl.pallas_call(
        _tc_kernel,
        grid=(n_i, n_j),
        in_specs=[
            pl.BlockSpec((TC_ROWS, TC_COLS), lambda i, j: (i, j)),
            pl.BlockSpec((1, 1, TC_ROWS), lambda i, j: (i, 0, 0)),
        ],
        out_specs=[
            pl.BlockSpec((1, 1), lambda i, j: (0, 0)),
            pl.BlockSpec((1, 1), lambda i, j: (0, 0)),
        ],
        out_shape=[
            jax.ShapeDtypeStruct((1, 1), jnp.float32),
            jax.ShapeDtypeStruct((1, 1), jnp.float32),
        ],
    )(x, m3)

    sc_gather = _make_sc_gather(N, V)
    gpart = sc_gather(x, t, m)
    g = jnp.sum(gpart)

    return (jnp.float32(c_const)
            - (fill * acc[0, 0] + dconf * g) / msum[0, 0])


# speedup vs baseline: 2.7582x; 1.0000x over previous
import functools
import math

import jax
import jax.numpy as jnp
import numpy as np
from jax import lax
from jax.experimental import pallas as pl
from jax.experimental.pallas import tpu as pltpu
from jax.experimental.pallas import tpu_sc as plsc

SMOOTHING = 0.1
CONFIDENCE = 1.0 - SMOOTHING

TC_ROWS = 1024
TC_COLS = 4096


def _tc_kernel(x_ref, m_ref, acc_ref, msum_ref):
    i = pl.program_id(0)
    j = pl.program_id(1)

    @pl.when((i == 0) & (j == 0))
    def _init():
        acc_ref[...] = jnp.zeros((1, 1), jnp.float32)
        msum_ref[...] = jnp.zeros((1, 1), jnp.float32)

    xb = x_ref[...]
    mb = m_ref[0, 0, :]
    rsum = jnp.sum(xb, axis=1)
    acc_ref[...] += jnp.sum(rsum * mb).reshape(1, 1)

    @pl.when(j == 0)
    def _msum():
        msum_ref[...] += jnp.sum(mb).reshape(1, 1)


def _make_sc_gather(N, V):
    mesh = plsc.VectorSubcoreMesh(core_axis_name="c", subcore_axis_name="s")
    NC = 2
    L = 16
    NW = 32
    rows_per_w = N // NW
    n_chunks = rows_per_w // L

    @functools.partial(
        pl.kernel,
        mesh=mesh,
        out_type=jax.ShapeDtypeStruct((NW, L), jnp.float32),
        scratch_types=[
            pltpu.VMEM((rows_per_w,), jnp.int32),
            pltpu.VMEM((rows_per_w,), jnp.float32),
            pltpu.VMEM((rows_per_w * L,), jnp.float32),
            pltpu.VMEM((L,), jnp.float32),
            pltpu.SemaphoreType.DMA,
        ],
    )
    def sc_gather(x_hbm, t_hbm, m_hbm, out_hbm, t_v, m_v, seg_v, stage_v,
                  sem):
        wid = lax.axis_index("s") * NC + lax.axis_index("c")
        row0 = wid * rows_per_w
        pltpu.sync_copy(t_hbm.at[pl.ds(row0, rows_per_w)], t_v)
        pltpu.sync_copy(m_hbm.at[pl.ds(row0, rows_per_w)], m_v)

        iota = lax.iota(jnp.int32, L)
        acc = jnp.zeros((L,), jnp.float32)

        t_i = []
        copies = []
        for c in range(n_chunks):
            tv = t_v[pl.ds(c * L, L)]
            t_i.extend(tv[i] for i in range(L))
        for r in range(rows_per_w):
            c0 = pl.multiple_of(t_i[r] & ~(L - 1), L)
            copies.append(pltpu.async_copy(
                x_hbm.at[row0 + r, pl.ds(c0, L)],
                seg_v.at[pl.ds(r * L, L)], sem))
        for cp in copies:
            cp.wait()
        for c in range(n_chunks):
            mv = m_v[pl.ds(c * L, L)]
            for i in range(L):
                r = c * L + i
                seg = seg_v[pl.ds(r * L, L)]
                hit = iota == (t_i[r] & (L - 1))
                acc = acc + jnp.where(hit, seg, 0.0) * mv[i]

        stage_v[...] = acc
        pltpu.sync_copy(stage_v, out_hbm.at[wid])

    return sc_gather


def kernel(input, target, mask):
    B, T, V = input.shape
    N = B * T
    x = input.reshape(N, V)
    t = target.reshape(N).astype(jnp.int32)
    m = mask.reshape(N).astype(jnp.float32)

    fill = float(np.float32(SMOOTHING / (V - 1)))
    conf = CONFIDENCE
    dconf = float(np.float32(conf - fill))
    c_const = (V - 1) * fill * math.log(fill) + conf * math.log(conf)

    n_i = N // TC_ROWS
    n_j = V // TC_COLS
    m3 = m.reshape(n_i, 1, TC_ROWS)
    acc, msum = pl.pallas_call(
        _tc_kernel,
        grid=(n_i, n_j),
        in_specs=[
            pl.BlockSpec((TC_ROWS, TC_COLS), lambda i, j: (i, j)),
            pl.BlockSpec((1, 1, TC_ROWS), lambda i, j: (i, 0, 0)),
        ],
        out_specs=[
            pl.BlockSpec((1, 1), lambda i, j: (0, 0)),
            pl.BlockSpec((1, 1), lambda i, j: (0, 0)),
        ],
        out_shape=[
            jax.ShapeDtypeStruct((1, 1), jnp.float32),
            jax.ShapeDtypeStruct((1, 1), jnp.float32),
        ],
    )(x, m3)

    sc_gather = _make_sc_gather(N, V)
    gpart = sc_gather(x, t, m)
    g = jnp.sum(gpart)

    return (jnp.float32(c_const)
            - (fill * acc[0, 0] + dconf * g) / msum[0, 0])


# speedup vs baseline: 2.8025x; 1.0161x over previous
import functools
import math

import jax
import jax.numpy as jnp
import numpy as np
from jax import lax
from jax.experimental import pallas as pl
from jax.experimental.pallas import tpu as pltpu
from jax.experimental.pallas import tpu_sc as plsc

SMOOTHING = 0.1
CONFIDENCE = 1.0 - SMOOTHING

TC_ROWS = 1024
TC_COLS = 4096


def _tc_kernel(x_ref, m_ref, acc_ref, msum_ref):
    i = pl.program_id(0)
    j = pl.program_id(1)

    @pl.when((i == 0) & (j == 0))
    def _init():
        acc_ref[...] = jnp.zeros((1, 1), jnp.float32)
        msum_ref[...] = jnp.zeros((1, 1), jnp.float32)

    xb = x_ref[...]
    mb = m_ref[0, 0, :]
    rsum = jnp.sum(xb, axis=1)
    acc_ref[...] += jnp.sum(rsum * mb).reshape(1, 1)

    @pl.when(j == 0)
    def _msum():
        msum_ref[...] += jnp.sum(mb).reshape(1, 1)


def _make_sc_gather(N, V):
    mesh = plsc.VectorSubcoreMesh(core_axis_name="c", subcore_axis_name="s",
                                  num_cores=1)
    NC = 1
    L = 16
    NW = 16
    rows_per_w = N // NW
    n_chunks = rows_per_w // L

    @functools.partial(
        pl.kernel,
        mesh=mesh,
        out_type=jax.ShapeDtypeStruct((NW, L), jnp.float32),
        scratch_types=[
            pltpu.VMEM((rows_per_w,), jnp.int32),
            pltpu.VMEM((rows_per_w,), jnp.float32),
            pltpu.VMEM((rows_per_w * L,), jnp.float32),
            pltpu.VMEM((L,), jnp.float32),
            pltpu.SemaphoreType.DMA,
        ],
    )
    def sc_gather(x_hbm, t_hbm, m_hbm, out_hbm, t_v, m_v, seg_v, stage_v,
                  sem):
        wid = lax.axis_index("s") * NC + lax.axis_index("c")
        row0 = wid * rows_per_w
        pltpu.sync_copy(t_hbm.at[pl.ds(row0, rows_per_w)], t_v)
        pltpu.sync_copy(m_hbm.at[pl.ds(row0, rows_per_w)], m_v)

        iota = lax.iota(jnp.int32, L)
        acc = jnp.zeros((L,), jnp.float32)

        t_i = []
        copies = []
        for c in range(n_chunks):
            tv = t_v[pl.ds(c * L, L)]
            t_i.extend(tv[i] for i in range(L))
        for r in range(rows_per_w):
            c0 = pl.multiple_of(t_i[r] & ~(L - 1), L)
            copies.append(pltpu.async_copy(
                x_hbm.at[row0 + r, pl.ds(c0, L)],
                seg_v.at[pl.ds(r * L, L)], sem))
        for cp in copies:
            cp.wait()
        for c in range(n_chunks):
            mv = m_v[pl.ds(c * L, L)]
            for i in range(L):
                r = c * L + i
                seg = seg_v[pl.ds(r * L, L)]
                hit = iota == (t_i[r] & (L - 1))
                acc = acc + jnp.where(hit, seg, 0.0) * mv[i]

        stage_v[...] = acc
        pltpu.sync_copy(stage_v, out_hbm.at[wid])

    return sc_gather


def kernel(input, target, mask):
    B, T, V = input.shape
    N = B * T
    x = input.reshape(N, V)
    t = target.reshape(N).astype(jnp.int32)
    m = mask.reshape(N).astype(jnp.float32)

    fill = float(np.float32(SMOOTHING / (V - 1)))
    conf = CONFIDENCE
    dconf = float(np.float32(conf - fill))
    c_const = (V - 1) * fill * math.log(fill) + conf * math.log(conf)

    n_i = N // TC_ROWS
    n_j = V // TC_COLS
    m3 = m.reshape(n_i, 1, TC_ROWS)
    acc, msum = pl.pallas_call(
        _tc_kernel,
        grid=(n_i, n_j),
        in_specs=[
            pl.BlockSpec((TC_ROWS, TC_COLS), lambda i, j: (i, j)),
            pl.BlockSpec((1, 1, TC_ROWS), lambda i, j: (i, 0, 0)),
        ],
        out_specs=[
            pl.BlockSpec((1, 1), lambda i, j: (0, 0)),
            pl.BlockSpec((1, 1), lambda i, j: (0, 0)),
        ],
        out_shape=[
            jax.ShapeDtypeStruct((1, 1), jnp.float32),
            jax.ShapeDtypeStruct((1, 1), jnp.float32),
        ],
    )(x, m3)

    sc_gather = _make_sc_gather(N, V)
    gpart = sc_gather(x, t, m)
    g = jnp.sum(gpart)

    return (jnp.float32(c_const)
            - (fill * acc[0, 0] + dconf * g) / msum[0, 0])
